# baseline (speedup 1.0000x reference)
import functools

import jax
import jax.numpy as jnp
from jax.experimental import pallas as pl
from jax.experimental.pallas import tpu as pltpu

_N = 10000
_D = 128
_K = 50
_MAXIT = 20
_ALPHA = 0.99
_C = 100


def _argmax_body(z_ref, out_ref):
    z = z_ref[...]
    m = jnp.max(z, axis=1, keepdims=True)
    ids = jax.lax.broadcasted_iota(jnp.int32, z.shape, 1)
    idx = jnp.min(jnp.where(z == m, ids, _C), axis=1)
    out_ref[...] = jnp.broadcast_to(idx[:, None], z.shape).astype(jnp.int32)


def _p_labels_pallas(Z):
    Zc = jnp.maximum(Z, 0.0)
    Zp = jnp.pad(Zc, ((0, 0), (0, 128 - _C)), constant_values=-jnp.inf)
    out = pl.pallas_call(
        _argmax_body,
        out_shape=jax.ShapeDtypeStruct((_N, 128), jnp.int32),
    )(Zp)
    return out[:, 0]


def kernel(X, labels, labels_mask, idxs):
    Xn = X / jnp.clip(jnp.linalg.norm(X, axis=1, keepdims=True), 1e-12)
    sims = Xn @ Xn.T
    Dv, Iv = jax.lax.top_k(sims, _K + 1)
    Dv = Dv[:, 1:] ** 3
    Iv = Iv[:, 1:]
    rows = jnp.broadcast_to(jnp.arange(_N)[:, None], (_N, _K)).reshape(-1)
    cols = Iv.reshape(-1)
    vals = Dv.reshape(-1)
    diag_w0 = jax.ops.segment_sum(jnp.where(rows == cols, vals, 0.0), rows,
                                  num_segments=_N)
    S = (jax.ops.segment_sum(vals, rows, num_segments=_N)
         + jax.ops.segment_sum(vals, cols, num_segments=_N)
         - 2.0 * diag_w0)
    S = jnp.where(S == 0.0, 1.0, S)
    Dn = 1.0 / jnp.sqrt(S)

    def W_mat(V):
        a = jax.ops.segment_sum(vals[:, None] * V[cols], rows, num_segments=_N)
        b = jax.ops.segment_sum(vals[:, None] * V[rows], cols, num_segments=_N)
        return a + b - 2.0 * diag_w0[:, None] * V

    def A_mat(V):
        return V - _ALPHA * (Dn[:, None] * W_mat(Dn[:, None] * V))

    counts = jax.ops.segment_sum(labels_mask.astype(jnp.int32), labels,
                                 num_segments=_C)
    seed_vals = jnp.where(labels_mask, 1.0 / counts[labels].astype(jnp.float32), 0.0)
    Y = jnp.zeros((_N, _C), dtype=jnp.float32).at[idxs, labels].set(seed_vals)

    Xc = jnp.zeros_like(Y)
    R = Y - A_mat(Xc)
    P = R
    rs = jnp.sum(R * R, axis=0)
    for _ in range(_MAXIT):
        AP = A_mat(P)
        alpha_c = rs / jnp.clip(jnp.sum(P * AP, axis=0), 1e-30)
        Xc = Xc + alpha_c * P
        R = R - alpha_c * AP
        rs_new = jnp.sum(R * R, axis=0)
        P = R + (rs_new / jnp.clip(rs, 1e-30)) * P
        rs = rs_new
    Z = Xc

    p_labels = _p_labels_pallas(Z)
    acc = jnp.mean((p_labels == labels).astype(jnp.float32))
    p_labels = jnp.where(labels_mask, labels.astype(p_labels.dtype), p_labels)
    return p_labels, acc
